# baseline (speedup 1.0000x reference)
import functools

import jax
import jax.numpy as jnp
from jax import lax
from jax.experimental import pallas as pl
from jax.experimental.pallas import tpu as pltpu
from jax.experimental.pallas import tpu_sc as plsc

_B = 8
_MAX_LEN = 2048
_NW = 16
_CHUNK = _B * _MAX_LEN // _NW
_VPW = _CHUNK // 16


def _sc_body(pieces_hbm, len_hbm, out_hbm, rs_hbm, len_v, buf_v, rs_v, s0, s1):
    wid = lax.axis_index("s")
    base = wid * _CHUNK
    row = base // _MAX_LEN
    col0 = base % _MAX_LEN

    c_len = pltpu.async_copy(len_hbm, len_v.at[pl.ds(0, _B)], s0)
    c_buf = pltpu.async_copy(pieces_hbm.at[pl.ds(base, _CHUNK)], buf_v, s1)
    c_len.wait()
    c_buf.wait()

    lane = lax.broadcasted_iota(jnp.int32, (16,), 0)
    lv = jnp.where(lane < _B, len_v[...], 0)
    row_len = jnp.sum(jnp.where(lane == row, lv, 0))
    @plsc.parallel_loop(0, _CHUNK, step=16, unroll=8)
    def _mask(off):
        col = lane + (col0 + off)
        vals = buf_v[pl.ds(off, 16)]
        buf_v[pl.ds(off, 16)] = jnp.where(col < row_len, vals, 0)

    pltpu.sync_copy(buf_v, out_hbm.at[pl.ds(base, _CHUNK)])

    @pl.when(wid == 0)
    def _():
        excl = plsc.cumsum(lv) - lv
        rs_v[...] = jnp.where(lane < _B, excl, jnp.sum(lv))
        pltpu.sync_copy(rs_v.at[pl.ds(0, _B + 1)], rs_hbm)


@functools.partial(jax.jit, static_argnames=())
def kernel(pieces, lengths):
    mesh = plsc.VectorSubcoreMesh(
        core_axis_name="c", subcore_axis_name="s", num_cores=1
    )
    flat = pieces.reshape(_B * _MAX_LEN)
    out_flat, rs = pl.kernel(
        _sc_body,
        out_type=[
            jax.ShapeDtypeStruct((_B * _MAX_LEN,), jnp.int32),
            jax.ShapeDtypeStruct((_B + 1,), jnp.int32),
        ],
        mesh=mesh,
        scratch_types=[
            pltpu.VMEM((16,), jnp.int32),
            pltpu.VMEM((_CHUNK,), jnp.int32),
            pltpu.VMEM((16,), jnp.int32),
            pltpu.SemaphoreType.DMA,
            pltpu.SemaphoreType.DMA,
        ],
        compiler_params=pltpu.CompilerParams(
            needs_layout_passes=False,
            disable_bounds_checks=True,
        ),
    )(flat, lengths)
    return out_flat.reshape(_B, _MAX_LEN), rs


# speedup vs baseline: 1.0207x; 1.0207x over previous
import functools

import jax
import jax.numpy as jnp
from jax import lax
from jax.experimental import pallas as pl
from jax.experimental.pallas import tpu as pltpu
from jax.experimental.pallas import tpu_sc as plsc

_B = 8
_MAX_LEN = 2048
_NW = 16
_CHUNK = _B * _MAX_LEN // _NW
_VPW = _CHUNK // 16


def _sc_body(pieces_hbm, len_hbm, out_hbm, rs_hbm, len_v, buf_v, rs_v, s0, s1):
    wid = lax.axis_index("s")
    base = wid * _CHUNK
    row = base // _MAX_LEN
    col0 = base % _MAX_LEN

    c_len = pltpu.async_copy(len_hbm, len_v.at[pl.ds(0, _B)], s0)
    c_buf = pltpu.async_copy(pieces_hbm.at[pl.ds(base, _CHUNK)], buf_v, s1)
    c_len.wait()
    c_buf.wait()

    lane = lax.broadcasted_iota(jnp.int32, (16,), 0)
    lv = jnp.where(lane < _B, len_v[...], 0)
    row_len = jnp.sum(jnp.where(lane == row, lv, 0))
    for j in range(_VPW):
        col = lane + (col0 + j * 16)
        vals = buf_v[pl.ds(j * 16, 16)]
        buf_v[pl.ds(j * 16, 16)] = jnp.where(col < row_len, vals, 0)

    pltpu.sync_copy(buf_v, out_hbm.at[pl.ds(base, _CHUNK)])

    @pl.when(wid == 0)
    def _():
        excl = plsc.cumsum(lv) - lv
        rs_v[...] = jnp.where(lane < _B, excl, jnp.sum(lv))
        pltpu.sync_copy(rs_v.at[pl.ds(0, _B + 1)], rs_hbm)


@functools.partial(jax.jit, static_argnames=())
def kernel(pieces, lengths):
    mesh = plsc.VectorSubcoreMesh(
        core_axis_name="c", subcore_axis_name="s", num_cores=1
    )
    flat = pieces.reshape(_B * _MAX_LEN)
    out_flat, rs = pl.kernel(
        _sc_body,
        out_type=[
            jax.ShapeDtypeStruct((_B * _MAX_LEN,), jnp.int32),
            jax.ShapeDtypeStruct((_B + 1,), jnp.int32),
        ],
        mesh=mesh,
        scratch_types=[
            pltpu.VMEM((16,), jnp.int32),
            pltpu.VMEM((_CHUNK,), jnp.int32),
            pltpu.VMEM((16,), jnp.int32),
            pltpu.SemaphoreType.DMA,
            pltpu.SemaphoreType.DMA,
        ],
        compiler_params=pltpu.CompilerParams(
            needs_layout_passes=False,
            disable_bounds_checks=True,
        ),
    )(flat, lengths)
    return out_flat.reshape(_B, _MAX_LEN), rs
